# baseline (speedup 1.0000x reference)
import functools

import jax
import jax.numpy as jnp
from jax import lax
from jax.experimental import pallas as pl
from jax.experimental.pallas import tpu as pltpu
from jax.experimental.pallas import tpu_sc as plsc

_MAX_PATH_DISTANCE = 5.0
_LANES = 16
_TILE = 80000
_F32_MAX = float(jnp.finfo(jnp.float32).max)


def kernel(x, edge_attr, weights, edge_vector):
    n_rows, n_cols = weights.shape
    total = n_rows * n_cols
    n_tiles = total // _TILE
    assert n_tiles * _TILE == total

    info = plsc.get_sparse_core_info()
    nc, ns = info.num_cores, info.num_subcores
    nw = nc * ns
    tiles_per_worker = pl.cdiv(n_tiles, nw)

    mesh = plsc.VectorSubcoreMesh(core_axis_name="c", subcore_axis_name="s")

    @functools.partial(
        pl.kernel,
        mesh=mesh,
        out_type=jax.ShapeDtypeStruct((total,), jnp.float32),
        scratch_types=[
            pltpu.VMEM((_LANES,), jnp.float32),
            pltpu.VMEM((_TILE,), jnp.float32),
        ],
    )
    def _sc_edge_encoding(scale_hbm, w_hbm, out_hbm, scale_buf, buf):
        wid = lax.axis_index("s") * nc + lax.axis_index("c")
        pltpu.sync_copy(scale_hbm, scale_buf)
        scale = scale_buf[...]

        def do_tile(t):
            off = t * _TILE
            pltpu.sync_copy(w_hbm.at[pl.ds(off, _TILE)], buf)

            def vec_step(i, carry):
                base = i * _LANES
                v = buf[pl.ds(base, _LANES)]
                r = jnp.minimum(v, _MAX_PATH_DISTANCE) * scale
                r_clipped = jnp.minimum(jnp.maximum(r, -_F32_MAX), _F32_MAX)
                buf[pl.ds(base, _LANES)] = jnp.where(r != r, 0.0, r_clipped)
                return carry

            lax.fori_loop(0, _TILE // _LANES, vec_step, 0, unroll=8)
            pltpu.sync_copy(buf, out_hbm.at[pl.ds(off, _TILE)])

        def tile_step(k, carry):
            t = wid + k * nw

            @pl.when(t < n_tiles)
            def _():
                do_tile(t)

            return carry

        lax.fori_loop(0, tiles_per_worker, tile_step, 0)

    scale_vec = jnp.broadcast_to(jnp.mean(edge_vector), (_LANES,))
    out_flat = _sc_edge_encoding(scale_vec, weights.reshape(total))
    return out_flat.reshape(n_rows, n_cols)


# speedup vs baseline: 4.7130x; 4.7130x over previous
import jax
import jax.numpy as jnp
from jax import lax
from jax.experimental import pallas as pl
from jax.experimental.pallas import tpu as pltpu

_MAX_PATH_DISTANCE = 5.0
_R = 200
_NIN = 4
_NOUT = 3


def _body(ev_ref, w_hbm, o_hbm, in_buf, out_buf, in_sems, out_sems):
    n_rows = w_hbm.shape[0]
    n_chunks = n_rows // _R
    s = jnp.sum(ev_ref[...]) / ev_ref.size

    def in_copy(c, slot):
        return pltpu.make_async_copy(
            w_hbm.at[pl.ds(c * _R, _R), :],
            in_buf.at[pl.ds(slot * _R, _R), :],
            in_sems.at[slot],
        )

    def out_copy(c, slot):
        return pltpu.make_async_copy(
            out_buf.at[pl.ds(slot * _R, _R), :],
            o_hbm.at[pl.ds(c * _R, _R), :],
            out_sems.at[slot],
        )

    for c in range(_NIN):
        in_copy(c, c).start()

    def step(c, carry):
        islot = lax.rem(c, _NIN)
        oslot = lax.rem(c, _NOUT)
        in_copy(c, islot).wait()

        @pl.when(c >= _NOUT)
        def _():
            out_copy(c - _NOUT, oslot).wait()

        v = in_buf[pl.ds(islot * _R, _R), :]
        out_buf[pl.ds(oslot * _R, _R), :] = jnp.nan_to_num(
            jnp.minimum(v, jnp.float32(_MAX_PATH_DISTANCE)) * s
        )
        out_copy(c, oslot).start()

        @pl.when(c + _NIN < n_chunks)
        def _():
            in_copy(c + _NIN, islot).start()

        return carry

    lax.fori_loop(0, n_chunks, step, 0)
    for k in range(_NOUT):
        c = n_chunks - _NOUT + k
        out_copy(c, c % _NOUT).wait()


def kernel(x, edge_attr, weights, edge_vector):
    n_rows, n_cols = weights.shape
    return pl.pallas_call(
        _body,
        in_specs=[
            pl.BlockSpec(edge_vector.shape, lambda: (0, 0)),
            pl.BlockSpec(memory_space=pltpu.MemorySpace.HBM),
        ],
        out_specs=pl.BlockSpec(memory_space=pltpu.MemorySpace.HBM),
        out_shape=jax.ShapeDtypeStruct((n_rows, n_cols), jnp.float32),
        scratch_shapes=[
            pltpu.VMEM((_NIN * _R, n_cols), jnp.float32),
            pltpu.VMEM((_NOUT * _R, n_cols), jnp.float32),
            pltpu.SemaphoreType.DMA((_NIN,)),
            pltpu.SemaphoreType.DMA((_NOUT,)),
        ],
    )(edge_vector, weights)
